# Initial kernel scaffold; baseline (speedup 1.0000x reference)
import jax
import jax.numpy as jnp
from jax.experimental import pallas as pl


def kernel(formula_vectors, atom_w, count_w, pos_w, ln_w, ln_b):
    raise NotImplementedError("write your pallas kernel here")


# speedup vs baseline: 4.2288x; 4.2288x over previous
import functools

import jax
import jax.numpy as jnp
from jax import lax
from jax.experimental import pallas as pl
from jax.experimental.pallas import tpu as pltpu
from jax.experimental.pallas import tpu_sc as plsc

B = 4096
A = 30
D = 768
MC = 200
EPS = 1e-5
S = 208
NIDX = B * A
W = 64


def _table_body(atom_ref, pos_ref, cw_ref, lnw_ref, lnb_ref, out_ref):
    base = atom_ref[0, 0, :] + pos_ref[0, 0, :]
    x = cw_ref[...] + base[None, :]
    mean = jnp.mean(x, axis=1, keepdims=True)
    xc = x - mean
    var = jnp.mean(xc * xc, axis=1, keepdims=True)
    y = xc * lax.rsqrt(var + EPS) * lnw_ref[0, :][None, :] + lnb_ref[0, :][None, :]
    out_ref[0] = y


def _idx_mask_body(f_ref, idx_ref, mask_ref):
    f = f_ref[...]
    c = jnp.clip(f, 0, MC)
    a_ids = lax.broadcasted_iota(jnp.int32, f.shape, 1)
    idx_ref[...] = a_ids * S + c
    mask_ref[...] = (f > 0).astype(jnp.float32)


def _build_table(atom_w, pos_w, count_w, ln_w, ln_b):
    cw_pad = jnp.pad(count_w, ((0, S - (MC + 1)), (0, 0)))
    return pl.pallas_call(
        _table_body,
        grid=(A,),
        in_specs=[
            pl.BlockSpec((1, 1, D), lambda a: (a, 0, 0)),
            pl.BlockSpec((1, 1, D), lambda a: (a, 0, 0)),
            pl.BlockSpec((S, D), lambda a: (0, 0)),
            pl.BlockSpec((1, D), lambda a: (0, 0)),
            pl.BlockSpec((1, D), lambda a: (0, 0)),
        ],
        out_specs=pl.BlockSpec((1, S, D), lambda a: (a, 0, 0)),
        out_shape=jax.ShapeDtypeStruct((A, S, D), jnp.float32),
    )(atom_w.reshape(A, 1, D), pos_w.reshape(A, 1, D), cw_pad,
      ln_w.reshape(1, D), ln_b.reshape(1, D))


def _build_idx_mask(formula_vectors):
    BB = 512
    return pl.pallas_call(
        _idx_mask_body,
        grid=(B // BB,),
        in_specs=[pl.BlockSpec((BB, A), lambda i: (i, 0))],
        out_specs=[
            pl.BlockSpec((BB, A), lambda i: (i, 0)),
            pl.BlockSpec((BB, A), lambda i: (i, 0)),
        ],
        out_shape=[
            jax.ShapeDtypeStruct((B, A), jnp.int32),
            jax.ShapeDtypeStruct((B, A), jnp.float32),
        ],
    )(formula_vectors)


@functools.cache
def _make_sc_gather():
    @functools.partial(
        pl.kernel,
        out_type=jax.ShapeDtypeStruct((NIDX, D), jnp.float32),
        mesh=plsc.VectorSubcoreMesh(core_axis_name="c", subcore_axis_name="s"),
    )
    def _sc_gather(table_hbm, idx_hbm, out_hbm):
        def body(i_vmem, o_vmem):
            pltpu.sync_copy(table_hbm.at[i_vmem.at[0, 0]], o_vmem)

        pltpu.emit_pipeline(
            body,
            grid=(NIDX // W,),
            in_specs=[pl.BlockSpec((1, 1, W), lambda i: (i, 0, 0))],
            out_specs=[pl.BlockSpec((W, D), lambda i: (i, 0))],
            core_axis_name=("c", "s"),
            dimension_semantics=(pltpu.PARALLEL,),
        )(idx_hbm, out_hbm)

    return _sc_gather


def kernel(formula_vectors, atom_w, count_w, pos_w, ln_w, ln_b):
    table = _build_table(atom_w, pos_w, count_w, ln_w, ln_b)
    idx, mask = _build_idx_mask(formula_vectors)
    rows = _make_sc_gather()(table.reshape(A * S, D), idx.reshape(NIDX // W, 1, W))
    return rows.reshape(B, A, D), mask


# speedup vs baseline: 12.4195x; 2.9369x over previous
import functools

import jax
import jax.numpy as jnp
from jax import lax
from jax.experimental import pallas as pl
from jax.experimental.pallas import tpu as pltpu
from jax.experimental.pallas import tpu_sc as plsc

B = 4096
A = 30
D = 768
MC = 200
EPS = 1e-5
S = 208
NIDX = B * A
W = 64


def _table_body(atom_ref, pos_ref, cw_ref, lnw_ref, lnb_ref, out_ref):
    base = atom_ref[0, 0, :] + pos_ref[0, 0, :]
    x = cw_ref[...] + base[None, :]
    mean = jnp.mean(x, axis=1, keepdims=True)
    xc = x - mean
    var = jnp.mean(xc * xc, axis=1, keepdims=True)
    y = xc * lax.rsqrt(var + EPS) * lnw_ref[0, :][None, :] + lnb_ref[0, :][None, :]
    out_ref[0] = y


def _idx_mask_body(ft_ref, f_ref, idx_ref, mask_ref):
    ft = ft_ref[...]
    c = jnp.clip(ft, 0, MC)
    a_ids = lax.broadcasted_iota(jnp.int32, ft.shape, 0)
    idx_ref[...] = a_ids * S + c
    mask_ref[...] = (f_ref[...] > 0).astype(jnp.float32)


def _build_table(atom_w, pos_w, count_w, ln_w, ln_b):
    cw_pad = jnp.pad(count_w, ((0, S - (MC + 1)), (0, 0)))
    return pl.pallas_call(
        _table_body,
        grid=(A,),
        in_specs=[
            pl.BlockSpec((1, 1, D), lambda a: (a, 0, 0)),
            pl.BlockSpec((1, 1, D), lambda a: (a, 0, 0)),
            pl.BlockSpec((S, D), lambda a: (0, 0)),
            pl.BlockSpec((1, D), lambda a: (0, 0)),
            pl.BlockSpec((1, D), lambda a: (0, 0)),
        ],
        out_specs=pl.BlockSpec((1, S, D), lambda a: (a, 0, 0)),
        out_shape=jax.ShapeDtypeStruct((A, S, D), jnp.float32),
    )(atom_w.reshape(A, 1, D), pos_w.reshape(A, 1, D), cw_pad,
      ln_w.reshape(1, D), ln_b.reshape(1, D))


def _build_idx_mask(formula_vectors):
    BB = 512
    return pl.pallas_call(
        _idx_mask_body,
        grid=(B // BB,),
        in_specs=[
            pl.BlockSpec((A, BB), lambda i: (0, i)),
            pl.BlockSpec((BB, A), lambda i: (i, 0)),
        ],
        out_specs=[
            pl.BlockSpec((A, BB), lambda i: (0, i)),
            pl.BlockSpec((BB, A), lambda i: (i, 0)),
        ],
        out_shape=[
            jax.ShapeDtypeStruct((A, B), jnp.int32),
            jax.ShapeDtypeStruct((B, A), jnp.float32),
        ],
    )(formula_vectors.T, formula_vectors)


@functools.cache
def _make_sc_gather():
    @functools.partial(
        pl.kernel,
        out_type=jax.ShapeDtypeStruct((NIDX, D), jnp.float32),
        mesh=plsc.VectorSubcoreMesh(core_axis_name="c", subcore_axis_name="s"),
    )
    def _sc_gather(table_hbm, idx_hbm, out_hbm):
        def body(i_vmem, o_vmem):
            pltpu.sync_copy(table_hbm.at[i_vmem.at[0, 0]], o_vmem)

        pltpu.emit_pipeline(
            body,
            grid=(NIDX // W,),
            in_specs=[pl.BlockSpec((1, 1, W), lambda i: (i, 0, 0))],
            out_specs=[pl.BlockSpec((W, D), lambda i: (i, 0))],
            core_axis_name=("c", "s"),
            dimension_semantics=(pltpu.PARALLEL,),
        )(idx_hbm, out_hbm)

    return _sc_gather


def kernel(formula_vectors, atom_w, count_w, pos_w, ln_w, ln_b):
    table = _build_table(atom_w, pos_w, count_w, ln_w, ln_b)
    idx, mask = _build_idx_mask(formula_vectors)
    rows = _make_sc_gather()(table.reshape(A * S, D), idx.reshape(NIDX // W, 1, W))
    return rows.reshape(A, B, D).transpose(1, 0, 2), mask
